# baseline (speedup 1.0000x reference)
import jax
import jax.numpy as jnp
from jax.experimental import pallas as pl
from jax.experimental.pallas import tpu as pltpu

_BM = 1000
_BK = 1280


def _make_kernel(n, bm, bk, nk, pad):
    limit = n - (nk - 1) * bk

    def _fused_kernel(x_ref, a_ref, w1_ref, w2_ref, o_ref, xw1_s, hw2_s, acc_s):
        p = pl.program_id(0)
        i = pl.program_id(1)
        k = pl.program_id(2)

        @pl.when((p == 0) & (k == 0))
        def _phase0():
            if pad > 0:
                @pl.when(i == 0)
                def _zero_tails():
                    zeros = jnp.zeros((pad, xw1_s.shape[1]), jnp.bfloat16)
                    xw1_s[pl.ds(n, pad), :] = zeros
                    hw2_s[pl.ds(n, pad), :] = zeros

            xw1_s[pl.ds(i * bm, bm), :] = jax.lax.dot_general(
                x_ref[...],
                w1_ref[...],
                (((1,), (1,)), ((), ())),
                preferred_element_type=jnp.float32,
            ).astype(jnp.bfloat16)

        def masked_a(chunk):
            a = a_ref[...].astype(jnp.bfloat16)
            cols = jax.lax.broadcasted_iota(jnp.int32, a.shape, 1)
            return jnp.where((chunk < nk - 1) | (cols < limit), a, jnp.bfloat16(0))

        @pl.when(p == 1)
        def _phase1():
            part = jnp.dot(
                masked_a(k),
                xw1_s[pl.ds(k * bk, bk), :],
                preferred_element_type=jnp.float32,
            )

            @pl.when(k == 0)
            def _():
                acc_s[...] = part

            @pl.when(k > 0)
            def _():
                acc_s[...] += part

            @pl.when(k == nk - 1)
            def _():
                h1 = jnp.maximum(acc_s[...], 0.0).astype(jnp.bfloat16)
                hw2_s[pl.ds(i * bm, bm), :] = jax.lax.dot_general(
                    h1,
                    w2_ref[...],
                    (((1,), (1,)), ((), ())),
                    preferred_element_type=jnp.float32,
                ).astype(jnp.bfloat16)

        @pl.when(p == 2)
        def _phase2():
            kk = nk - 1 - k
            part = jnp.dot(
                masked_a(kk),
                hw2_s[pl.ds(kk * bk, bk), :],
                preferred_element_type=jnp.float32,
            )

            @pl.when(k == 0)
            def _():
                acc_s[...] = part

            @pl.when(k > 0)
            def _():
                acc_s[...] += part

            @pl.when(k == nk - 1)
            def _():
                acc = acc_s[...]
                m = jnp.max(acc, axis=1, keepdims=True)
                lse = jnp.log(jnp.sum(jnp.exp(acc - m), axis=1, keepdims=True))
                o_ref[...] = acc - m - lse

    return _fused_kernel


def kernel(x, adjacency, W1, W2):
    n, d_in = x.shape
    d_h = W1.shape[0]
    d_out = W2.shape[0]
    bm = min(_BM, n)
    bk = min(_BK, n)
    num_i = n // bm
    num_k = -(-n // bk)
    pad_n = num_k * bk
    last_i = num_i - 1
    last_k = num_k - 1

    x_b = x.astype(jnp.bfloat16)
    w1_b = W1.astype(jnp.bfloat16)
    w2_b = W2.astype(jnp.bfloat16)

    def x_map(p, i, k):
        return (jnp.where(p == 0, i, last_i), 0)

    def a_map(p, i, k):
        return (
            jnp.where(p == 0, 0, jnp.where(p == 1, i, last_i - i)),
            jnp.where(p == 0, 0, jnp.where(p == 1, k, last_k - k)),
        )

    def o_map(p, i, k):
        return (jnp.where(p == 2, last_i - i, last_i), 0)

    out = pl.pallas_call(
        _make_kernel(n, bm, bk, num_k, pad_n - n),
        grid=(3, num_i, num_k),
        in_specs=[
            pl.BlockSpec((bm, d_in), x_map),
            pl.BlockSpec((bm, bk), a_map),
            pl.BlockSpec((d_h, d_in), lambda p, i, k: (0, 0)),
            pl.BlockSpec((d_out, d_h), lambda p, i, k: (0, 0)),
        ],
        out_specs=pl.BlockSpec((bm, d_out), o_map),
        out_shape=jax.ShapeDtypeStruct((n, d_out), jnp.float32),
        scratch_shapes=[
            pltpu.VMEM((pad_n, d_h), jnp.bfloat16),
            pltpu.VMEM((pad_n, d_out), jnp.bfloat16),
            pltpu.VMEM((bm, d_out), jnp.float32),
        ],
        compiler_params=pltpu.CompilerParams(
            dimension_semantics=("arbitrary", "arbitrary", "arbitrary"),
        ),
    )(x_b, adjacency, w1_b, w2_b)

    return out


# speedup vs baseline: 1.3335x; 1.3335x over previous
import jax
import jax.numpy as jnp
from jax.experimental import pallas as pl
from jax.experimental.pallas import tpu as pltpu

_BM = 400
_S = 2


def _make_kernel(s):
    def _fused_kernel(*args):
        x_ref = args[0]
        a_refs = args[1:1 + s]
        w1_ref, w2_ref, o_ref, xw1_s, hw2_s = args[1 + s:]
        p = pl.program_id(0)
        i = pl.program_id(1)
        bm = x_ref.shape[0]

        def a_tile():
            if s == 1:
                return a_refs[0][...].astype(jnp.bfloat16)
            return jnp.concatenate(
                [r[...].astype(jnp.bfloat16) for r in a_refs], axis=0
            )

        @pl.when(p == 0)
        def _phase0():
            xw1_s[pl.ds(i * bm, bm), :] = jax.lax.dot_general(
                x_ref[...],
                w1_ref[...],
                (((1,), (1,)), ((), ())),
                preferred_element_type=jnp.float32,
            ).astype(jnp.bfloat16)

        @pl.when(p == 1)
        def _phase1():
            acc = jnp.dot(
                a_tile(),
                xw1_s[...],
                preferred_element_type=jnp.float32,
            )
            acc = jnp.maximum(acc, 0.0).astype(jnp.bfloat16)
            hw2_s[pl.ds(i * bm, bm), :] = jax.lax.dot_general(
                acc,
                w2_ref[...],
                (((1,), (1,)), ((), ())),
                preferred_element_type=jnp.float32,
            ).astype(jnp.bfloat16)

        @pl.when(p == 2)
        def _phase2():
            acc = jnp.dot(
                a_tile(),
                hw2_s[...],
                preferred_element_type=jnp.float32,
            )
            m = jnp.max(acc, axis=1, keepdims=True)
            lse = jnp.log(jnp.sum(jnp.exp(acc - m), axis=1, keepdims=True))
            o_ref[...] = acc - m - lse

    return _fused_kernel


def kernel(x, adjacency, W1, W2):
    n, d_in = x.shape
    d_h = W1.shape[0]
    d_out = W2.shape[0]
    bm = min(_BM, n)
    s = _S if bm % _S == 0 and (bm // _S) % 8 == 0 else 1
    stripe = bm // s
    num_i = n // bm
    last = num_i - 1

    x_b = x.astype(jnp.bfloat16)
    w1_b = W1.astype(jnp.bfloat16)
    w2_b = W2.astype(jnp.bfloat16)

    def x_map(p, i):
        return (jnp.where(p == 0, i, last), 0)

    def a_map(j):
        def m(p, i):
            row_tile = jnp.where(p == 1, i, jnp.where(p == 0, 0, last - i))
            return (row_tile * s + j, 0)
        return m

    def o_map(p, i):
        return (jnp.where(p == 2, last - i, last), 0)

    out = pl.pallas_call(
        _make_kernel(s),
        grid=(3, num_i),
        in_specs=[pl.BlockSpec((bm, d_in), x_map)]
        + [pl.BlockSpec((stripe, n), a_map(j)) for j in range(s)]
        + [
            pl.BlockSpec((d_h, d_in), lambda p, i: (0, 0)),
            pl.BlockSpec((d_out, d_h), lambda p, i: (0, 0)),
        ],
        out_specs=pl.BlockSpec((bm, d_out), o_map),
        out_shape=jax.ShapeDtypeStruct((n, d_out), jnp.float32),
        scratch_shapes=[
            pltpu.VMEM((n, d_h), jnp.bfloat16),
            pltpu.VMEM((n, d_out), jnp.bfloat16),
        ],
        compiler_params=pltpu.CompilerParams(
            dimension_semantics=("arbitrary", "arbitrary"),
        ),
    )(x_b, *([adjacency] * s), w1_b, w2_b)

    return out


# speedup vs baseline: 1.3666x; 1.0249x over previous
import jax
import jax.numpy as jnp
from jax.experimental import pallas as pl
from jax.experimental.pallas import tpu as pltpu

_BM = 400
_BMX = 1000


def _make_kernel(nx):
    def _fused_kernel(x_ref, a_ref, w1_ref, w2_ref, o_ref, xw1_s, hw2_s):
        p = pl.program_id(0)
        i = pl.program_id(1)
        bmx = x_ref.shape[0]

        @pl.when((p == 0) & (i < nx))
        def _phase0():
            xw1_s[pl.ds(i * bmx, bmx), :] = jax.lax.dot_general(
                x_ref[...],
                w1_ref[...],
                (((1,), (1,)), ((), ())),
                preferred_element_type=jnp.float32,
            ).astype(jnp.bfloat16)

        @pl.when(p == 1)
        def _phase1():
            acc = jnp.dot(
                a_ref[...].astype(jnp.bfloat16),
                xw1_s[...],
                preferred_element_type=jnp.float32,
            )
            acc = jnp.maximum(acc, 0.0).astype(jnp.bfloat16)
            bm = a_ref.shape[0]
            hw2_s[pl.ds(i * bm, bm), :] = jax.lax.dot_general(
                acc,
                w2_ref[...],
                (((1,), (1,)), ((), ())),
                preferred_element_type=jnp.float32,
            ).astype(jnp.bfloat16)

        @pl.when(p == 2)
        def _phase2():
            acc = jnp.dot(
                a_ref[...].astype(jnp.bfloat16),
                hw2_s[...],
                preferred_element_type=jnp.float32,
            )
            m = jnp.max(acc, axis=1, keepdims=True)
            lse = jnp.log(jnp.sum(jnp.exp(acc - m), axis=1, keepdims=True))
            o_ref[...] = acc - m - lse

    return _fused_kernel


def kernel(x, adjacency, W1, W2):
    n, d_in = x.shape
    d_h = W1.shape[0]
    d_out = W2.shape[0]
    bm = min(_BM, n)
    num_i = n // bm
    bmx = _BMX if (n % _BMX == 0 and n // _BMX <= num_i) else bm
    num_x = n // bmx
    last = num_i - 1
    last_x = num_x - 1

    x_b = x.astype(jnp.bfloat16)
    w1_b = W1.astype(jnp.bfloat16)
    w2_b = W2.astype(jnp.bfloat16)

    def x_map(p, i):
        return (jnp.where(p == 0, jnp.minimum(i, last_x), last_x), 0)

    def a_map(p, i):
        return (jnp.where(p == 0, 0, jnp.where(p == 1, i, last - i)), 0)

    def o_map(p, i):
        return (jnp.where(p == 2, last - i, last), 0)

    out = pl.pallas_call(
        _make_kernel(num_x),
        grid=(3, num_i),
        in_specs=[
            pl.BlockSpec((bmx, d_in), x_map),
            pl.BlockSpec((bm, n), a_map),
            pl.BlockSpec((d_h, d_in), lambda p, i: (0, 0)),
            pl.BlockSpec((d_out, d_h), lambda p, i: (0, 0)),
        ],
        out_specs=pl.BlockSpec((bm, d_out), o_map),
        out_shape=jax.ShapeDtypeStruct((n, d_out), jnp.float32),
        scratch_shapes=[
            pltpu.VMEM((n, d_h), jnp.bfloat16),
            pltpu.VMEM((n, d_out), jnp.bfloat16),
        ],
        compiler_params=pltpu.CompilerParams(
            dimension_semantics=("arbitrary", "arbitrary"),
        ),
    )(x_b, adjacency, w1_b, w2_b)

    return out
